# baseline (speedup 1.0000x reference)
import jax
import jax.numpy as jnp
from jax.experimental import pallas as pl
from jax.experimental.pallas import tpu as pltpu

B, S, A, F = 256, 16, 128, 16
H = 64
R = 4096
KB = R // A


def _fused_kernel(x_ref, w_ref, ones_ref, ho_ref, wcat_ref,
                  wl1a_ref, cb1a_ref, wl1h_ref, cb1h_ref,
                  wl2h_ref, wl2a_ref, h1ss_ref, wr2s_ref, cb2_ref,
                  out_ref):
    wcat = wcat_ref[...]
    one_row = jnp.ones((1, R), jnp.bfloat16)
    hs = jnp.zeros((H, R), jnp.float32)
    c = jnp.zeros((H, R), jnp.float32)
    x = x_ref[...]
    for t in range(S):
        xt = x[t * F:(t + 1) * F, :]
        cat = jnp.concatenate([xt, hs.astype(jnp.bfloat16), one_row], axis=0)
        g = jnp.dot(wcat, cat, preferred_element_type=jnp.float32)
        tt = jnp.tanh(g)
        ti = tt[0:H]
        tf = tt[H:2 * H]
        tg = tt[2 * H:3 * H]
        to = tt[3 * H:4 * H]
        c = 0.5 * (tf * c + c + ti * tg + tg)
        tc = jnp.tanh(c)
        hs = to * tc + tc
    hw = hs * w_ref[...]
    mean = jnp.dot(hw, ones_ref[...], preferred_element_type=jnp.float32)
    h1a = jnp.dot(wl1a_ref[...], mean, preferred_element_type=jnp.float32) + cb1a_ref[...]
    h1h = jnp.dot(wl1h_ref[...], ho_ref[0], preferred_element_type=jnp.float32) + cb1h_ref[...]
    cvec = jnp.dot(wr2s_ref[...], h1ss_ref[...], preferred_element_type=jnp.float32) + cb2_ref[...]
    h2 = (jnp.dot(wl2h_ref[...], h1h, preferred_element_type=jnp.float32)
          + jnp.dot(wl2a_ref[...], h1a, preferred_element_type=jnp.float32)
          + cvec)
    out_ref[...] = jnp.tanh(h2)[None]


def kernel(agent_obs, hideout_obs, timestep_obs, lstm_params, conv1_params, conv2_params, num_agents):
    NC = 2
    BC = B // NC
    xTs = [agent_obs[k * BC:(k + 1) * BC].astype(jnp.bfloat16)
           .transpose(1, 3, 0, 2).reshape(S * F, BC * A) for k in range(NC)]
    valid = jnp.arange(A)[None, :] < num_agents[:, None]
    cnt = jnp.clip(num_agents.astype(jnp.float32), 1.0, None)
    w = (0.5 * valid.astype(jnp.float32) / cnt[:, None]).reshape(1, B * A)
    gate_scale = jnp.concatenate(
        [jnp.full((2 * H,), 0.5), jnp.ones((H,)), jnp.full((H,), 0.5)]).astype(jnp.float32)
    bias = ((lstm_params["b_ih"] + lstm_params["b_hh"]) * gate_scale).reshape(4 * H, 1)
    wcat = jnp.concatenate(
        [lstm_params["W_ih"] * gate_scale[:, None],
         lstm_params["W_hh"] * (0.5 * gate_scale[:, None]),
         bias], axis=1).astype(jnp.bfloat16)
    onesblk = jnp.repeat(jnp.eye(KB, dtype=jnp.float32), A, axis=0)
    grid = (BC * A) // R
    hoT3 = hideout_obs.T.reshape(2, NC * grid, KB).transpose(1, 0, 2)
    c1, c2 = conv1_params, conv2_params
    wl1a = c1["agent|agent_summ"]["Wl"].T
    cb1a = (c1["agent|agent_summ"]["b"] + c1["state_summ|agent_summ"]["b"]).reshape(H, 1)
    wl1h = c1["hideout|hideout_summ"]["Wl"].T
    cb1h = (c1["hideout|hideout_summ"]["b"] + c1["state_summ|hideout_summ"]["b"]).reshape(H, 1)
    h1ss = (c1["hideout_summ|state_summ"]["b"] + c1["agent_summ|state_summ"]["b"]).reshape(H, 1)
    wl2h = c2["hideout_summ|state_summ"]["Wl"].T
    wl2a = c2["agent_summ|state_summ"]["Wl"].T
    wr2s = (c2["hideout_summ|state_summ"]["Wr"] + c2["agent_summ|state_summ"]["Wr"]).T
    cb2 = (c2["hideout_summ|state_summ"]["b"] + c2["agent_summ|state_summ"]["b"]).reshape(H, 1)

    full = lambda i: (0, 0)
    call = lambda: pl.pallas_call(
        _fused_kernel,
        grid=(grid,),
        in_specs=[
            pl.BlockSpec((S * F, R), lambda i: (0, i)),
            pl.BlockSpec((1, R), lambda i: (0, i)),
            pl.BlockSpec((R, KB), full),
            pl.BlockSpec((1, 2, KB), lambda i: (i, 0, 0)),
            pl.BlockSpec((4 * H, F + H + 1), full),
            pl.BlockSpec((H, H), full),
            pl.BlockSpec((H, 1), full),
            pl.BlockSpec((H, 2), full),
            pl.BlockSpec((H, 1), full),
            pl.BlockSpec((H, H), full),
            pl.BlockSpec((H, H), full),
            pl.BlockSpec((H, 1), full),
            pl.BlockSpec((H, H), full),
            pl.BlockSpec((H, 1), full),
        ],
        out_specs=pl.BlockSpec((1, H, KB), lambda i: (i, 0, 0)),
        out_shape=jax.ShapeDtypeStruct((grid, H, KB), jnp.float32),
    )
    parts = [call()(xTs[k], w[:, k * BC * A:(k + 1) * BC * A],
                    onesblk, hoT3[k * grid:(k + 1) * grid],
                    wcat, wl1a, cb1a, wl1h, cb1h,
                    wl2h, wl2a, h1ss, wr2s, cb2)
             for k in range(NC)]
    resT = jnp.concatenate(parts, axis=0)
    res = resT.transpose(0, 2, 1).reshape(B, H)
    return jnp.concatenate([res, hideout_obs, timestep_obs], axis=-1)


# speedup vs baseline: 1.0605x; 1.0605x over previous
import jax
import jax.numpy as jnp
from jax.experimental import pallas as pl
from jax.experimental.pallas import tpu as pltpu

B, S, A, F = 256, 16, 128, 16
H = 64
R = 8192
KB = R // A


def _fused_kernel(x_ref, w_ref, ones_ref, ho_ref, wcat_ref,
                  wl1a_ref, cb1a_ref, wl1h_ref, cb1h_ref,
                  wl2h_ref, wl2a_ref, h1ss_ref, wr2s_ref, cb2_ref,
                  out_ref):
    wcat = wcat_ref[...]
    one_row = jnp.ones((1, R), jnp.bfloat16)
    hs = jnp.zeros((H, R), jnp.float32)
    c = jnp.zeros((H, R), jnp.float32)
    x = x_ref[...]
    for t in range(S):
        xt = x[t * F:(t + 1) * F, :]
        cat = jnp.concatenate([xt, hs.astype(jnp.bfloat16), one_row], axis=0)
        g = jnp.dot(wcat, cat, preferred_element_type=jnp.float32)
        tt = jnp.tanh(g)
        ti = tt[0:H]
        tf = tt[H:2 * H]
        tg = tt[2 * H:3 * H]
        to = tt[3 * H:4 * H]
        c = 0.5 * (tf * c + c + ti * tg + tg)
        tc = jnp.tanh(c)
        hs = to * tc + tc
    hw = (hs * w_ref[...]).astype(jnp.bfloat16)
    mean = jnp.dot(hw, ones_ref[...], preferred_element_type=jnp.float32)
    h1a = jnp.dot(wl1a_ref[...], mean, preferred_element_type=jnp.float32) + cb1a_ref[...]
    h1h = jnp.dot(wl1h_ref[...], ho_ref[0], preferred_element_type=jnp.float32) + cb1h_ref[...]
    cvec = jnp.dot(wr2s_ref[...], h1ss_ref[...], preferred_element_type=jnp.float32) + cb2_ref[...]
    h2 = (jnp.dot(wl2h_ref[...], h1h, preferred_element_type=jnp.float32)
          + jnp.dot(wl2a_ref[...], h1a, preferred_element_type=jnp.float32)
          + cvec)
    out_ref[...] = jnp.tanh(h2)[None]


def kernel(agent_obs, hideout_obs, timestep_obs, lstm_params, conv1_params, conv2_params, num_agents):
    xT = agent_obs.astype(jnp.bfloat16).transpose(1, 3, 0, 2).reshape(S * F, B * A)
    valid = jnp.arange(A)[None, :] < num_agents[:, None]
    cnt = jnp.clip(num_agents.astype(jnp.float32), 1.0, None)
    w = (0.5 * valid.astype(jnp.float32) / cnt[:, None]).reshape(1, B * A)
    gate_scale = jnp.concatenate(
        [jnp.full((2 * H,), 0.5), jnp.ones((H,)), jnp.full((H,), 0.5)]).astype(jnp.float32)
    bias = ((lstm_params["b_ih"] + lstm_params["b_hh"]) * gate_scale).reshape(4 * H, 1)
    wcat = jnp.concatenate(
        [lstm_params["W_ih"] * gate_scale[:, None],
         lstm_params["W_hh"] * (0.5 * gate_scale[:, None]),
         bias], axis=1).astype(jnp.bfloat16)
    onesblk = jnp.repeat(jnp.eye(KB, dtype=jnp.bfloat16), A, axis=0)
    grid = (B * A) // R
    hoT3 = hideout_obs.T.reshape(2, grid, KB).transpose(1, 0, 2)
    c1, c2 = conv1_params, conv2_params
    wl1a = c1["agent|agent_summ"]["Wl"].T
    cb1a = (c1["agent|agent_summ"]["b"] + c1["state_summ|agent_summ"]["b"]).reshape(H, 1)
    wl1h = c1["hideout|hideout_summ"]["Wl"].T
    cb1h = (c1["hideout|hideout_summ"]["b"] + c1["state_summ|hideout_summ"]["b"]).reshape(H, 1)
    h1ss = (c1["hideout_summ|state_summ"]["b"] + c1["agent_summ|state_summ"]["b"]).reshape(H, 1)
    wl2h = c2["hideout_summ|state_summ"]["Wl"].T
    wl2a = c2["agent_summ|state_summ"]["Wl"].T
    wr2s = (c2["hideout_summ|state_summ"]["Wr"] + c2["agent_summ|state_summ"]["Wr"]).T
    cb2 = (c2["hideout_summ|state_summ"]["b"] + c2["agent_summ|state_summ"]["b"]).reshape(H, 1)

    full = lambda i: (0, 0)
    resT = pl.pallas_call(
        _fused_kernel,
        grid=(grid,),
        in_specs=[
            pl.BlockSpec((S * F, R), lambda i: (0, i)),
            pl.BlockSpec((1, R), lambda i: (0, i)),
            pl.BlockSpec((R, KB), full),
            pl.BlockSpec((1, 2, KB), lambda i: (i, 0, 0)),
            pl.BlockSpec((4 * H, F + H + 1), full),
            pl.BlockSpec((H, H), full),
            pl.BlockSpec((H, 1), full),
            pl.BlockSpec((H, 2), full),
            pl.BlockSpec((H, 1), full),
            pl.BlockSpec((H, H), full),
            pl.BlockSpec((H, H), full),
            pl.BlockSpec((H, 1), full),
            pl.BlockSpec((H, H), full),
            pl.BlockSpec((H, 1), full),
        ],
        out_specs=pl.BlockSpec((1, H, KB), lambda i: (i, 0, 0)),
        out_shape=jax.ShapeDtypeStruct((grid, H, KB), jnp.float32),
    )(xT, w, onesblk, hoT3, wcat, wl1a, cb1a, wl1h, cb1h,
      wl2h, wl2a, h1ss, wr2s, cb2)
    res = resT.transpose(0, 2, 1).reshape(B, H)
    return jnp.concatenate([res, hideout_obs, timestep_obs], axis=-1)
